# baseline (speedup 1.0000x reference)
import functools

import jax
import jax.numpy as jnp
from jax.experimental import pallas as pl
from jax.experimental.pallas import tpu as pltpu


def _add_kernel(in_ref, tab_ref, out_ref):
    out_ref[...] = in_ref[...] + tab_ref[...]


def _part(inputs, table, b0, nb, block=2048):
    batch, seq, dim = inputs.shape
    grid = (seq // block, nb)
    return pl.pallas_call(
        _add_kernel,
        grid=grid,
        in_specs=[
            pl.BlockSpec((1, block, dim), lambda s, b: (b0 + b, s, 0)),
            pl.BlockSpec((block, dim), lambda s, b: (s, 0)),
        ],
        out_specs=pl.BlockSpec((1, block, dim), lambda s, b: (b, s, 0)),
        out_shape=jax.ShapeDtypeStruct((nb, seq, dim), inputs.dtype),
    )(inputs, table)


@jax.jit
def _posemb_add(inputs, table):
    a = _part(inputs, table, 0, 3)
    b = _part(inputs, table, 3, 1)
    return jnp.concatenate([a, b], axis=0)


def kernel(inputs, table):
    return _posemb_add(inputs, table)
